# baseline (speedup 1.0000x reference)
import functools

import jax
import jax.numpy as jnp
from jax import lax
from jax.experimental import pallas as pl
from jax.experimental.pallas import tpu as pltpu
from jax.experimental.pallas import tpu_sc as plsc

B = 16384
F = 26
V = 100000
D = 32
C = 16
H0, H1, H2 = 512, 256, 128
FD = F * D

NC, NS = 2, 16
NW = NC * NS
TR = F * V
PR = TR // 4

K1_CH = 128
K1_RPW = 81280
K1_LAST = TR - (NW - 1) * K1_RPW

BPW = B // NW
IPW = BPW * F
GRP = 32
NG = BPW // GRP
NI = NG * F
RING = 8
K1R = 6

_mesh = plsc.VectorSubcoreMesh(core_axis_name="c", subcore_axis_name="s")


@functools.partial(
    pl.kernel,
    out_type=jax.ShapeDtypeStruct((PR, 128), jnp.float32),
    mesh=_mesh,
    scratch_types=(
        [pltpu.VMEM((K1_CH, 32), jnp.float32)] * K1R
        + [pltpu.VMEM((K1_CH // 4, 128), jnp.float32)] * K1R
        + [pltpu.SemaphoreType.DMA] * (2 * K1R)
    ),
)
def _k1(tab, out, *s):
    b32s, bps = s[0:K1R], s[K1R:2 * K1R]
    srs, sws = s[2 * K1R:3 * K1R], s[3 * K1R:4 * K1R]
    wid = lax.axis_index("s") * NC + lax.axis_index("c")
    wb = wid * K1_RPW
    rows_w = jnp.where(wid < NW - 1, K1_RPW, K1_LAST)
    end = wb + rows_w
    n_c = (rows_w + K1_CH - 1) // K1_CH
    n_blk = (n_c + K1R - 1) // K1R
    n_it = n_blk * K1R

    def base(c):
        return pl.multiple_of(jnp.minimum(wb + c * K1_CH, end - K1_CH), 32)

    for st in range(K1R):
        pltpu.async_copy(tab.at[pl.ds(base(st), K1_CH), :], b32s[st], srs[st])

    def blk_body(blk, carry):
        for st in range(K1R):
            b32, bp, sr, sw = b32s[st], bps[st], srs[st], sws[st]
            c = blk * K1R + st
            pltpu.make_async_copy(
                tab.at[pl.ds(base(c), K1_CH), :], b32, sr).wait()

            @pl.when(c >= K1R)
            def _():
                pltpu.make_async_copy(
                    bp, out.at[pl.ds(0, K1_CH // 4), :], sw).wait()

            def vbody(v, cy):
                for j in range(8):
                    bp[v, pl.ds(j * 16, 16)] = (
                        b32[4 * v + j // 2, pl.ds((j % 2) * 16, 16)])
                return cy

            lax.fori_loop(0, K1_CH // 4, vbody, 0)
            pltpu.async_copy(bp, out.at[pl.ds(pl.multiple_of(base(c) // 4, 8), K1_CH // 4), :], sw)

            @pl.when(c + K1R < n_it)
            def _():
                pltpu.async_copy(
                    tab.at[pl.ds(base(c + K1R), K1_CH), :], b32, sr)
        return carry

    lax.fori_loop(0, n_blk, blk_body, 0)
    for st in range(K1R):
        pltpu.make_async_copy(
            bps[st], out.at[pl.ds(0, K1_CH // 4), :], sws[st]).wait()


@functools.partial(
    pl.kernel,
    out_type=jax.ShapeDtypeStruct((B, FD), jnp.float32),
    mesh=_mesh,
    scratch_types=(
        [pltpu.VMEM((IPW,), jnp.int32)] * 2
        + [pltpu.VMEM((GRP, 128), jnp.float32)] * RING
        + [pltpu.VMEM((GRP, FD), jnp.float32)]
        + [pltpu.SemaphoreType.DMA] * (RING + 1)
    ),
    compiler_params=pltpu.CompilerParams(needs_layout_passes=False),
)
def _k2(tab128, quad, qb, out, *s):
    quad_v, qb_v = s[0], s[1]
    gbufs, asm = s[2:2 + RING], s[2 + RING]
    sems, sem_w = s[3 + RING:3 + 2 * RING], s[3 + 2 * RING]
    wid = lax.axis_index("s") * NC + lax.axis_index("c")
    wb = pl.multiple_of(wid * BPW, 32)
    ib = pl.multiple_of(wid * IPW, 8)
    pltpu.sync_copy(quad.at[pl.ds(ib, IPW)], quad_v)
    pltpu.sync_copy(qb.at[pl.ds(ib, IPW)], qb_v)

    def issue(f, g, buf, sem):
        pltpu.async_copy(
            tab128.at[quad_v.at[pl.ds(pl.multiple_of(f * BPW + g * GRP, 8), GRP)]], buf, sem)

    for st in range(RING):
        issue(st % F, st // F, gbufs[st], sems[st])

    iota = lax.iota(jnp.int32, 16)

    def blk_body(blk, carry):
        for st in range(RING):
            i = blk * RING + st
            f = i % F
            g = i // F
            buf, sem = gbufs[st], sems[st]
            pltpu.make_async_copy(
                tab128.at[quad_v.at[pl.ds(pl.multiple_of(f * BPW + g * GRP, 8), GRP)]],
                buf, sem).wait()
            @pl.when((f == 0) & (g > 0))
            def _():
                pltpu.make_async_copy(
                    asm, out.at[pl.ds(wb, GRP), :], sem_w).wait()

            for rg in range(2):
                rows = rg * 16 + iota
                qbv = plsc.load_gather(qb_v, [f * BPW + g * GRP + rows])
                for j in range(32):
                    v = plsc.load_gather(buf, [rows, qbv + j])
                    plsc.store_scatter(
                        asm, [rows, jnp.full((16,), f * 32 + j, jnp.int32)], v)

            @pl.when(i + RING < NI)
            def _():
                i2 = i + RING
                issue(i2 % F, i2 // F, buf, sem)

            @pl.when(f == F - 1)
            def _():
                pltpu.async_copy(
                    asm, out.at[pl.ds(pl.multiple_of(wb + g * GRP, 8), GRP), :], sem_w)
        return carry

    lax.fori_loop(0, NI // RING, blk_body, 0)
    pltpu.make_async_copy(asm, out.at[pl.ds(wb, GRP), :], sem_w).wait()


BM = 512


def _mlp_body(emb_ref, cont_ref, w0a_ref, w0b_ref, b0_ref, w1_ref, b1_ref,
              w2_ref, b2_ref, out_ref):
    h0 = jnp.dot(emb_ref[...], w0a_ref[...], preferred_element_type=jnp.float32)
    h0 += jnp.dot(cont_ref[...], w0b_ref[...], preferred_element_type=jnp.float32)
    h0 = jnp.maximum(h0 + b0_ref[...], 0.0)
    h1 = jnp.maximum(
        jnp.dot(h0, w1_ref[...], preferred_element_type=jnp.float32) + b1_ref[...], 0.0)
    out_ref[...] = jnp.maximum(
        jnp.dot(h1, w2_ref[...], preferred_element_type=jnp.float32) + b2_ref[...], 0.0)


_mlp = pl.pallas_call(
    _mlp_body,
    grid=(B // BM,),
    in_specs=[
        pl.BlockSpec((BM, FD), lambda i: (i, 0)),
        pl.BlockSpec((BM, C), lambda i: (i, 0)),
        pl.BlockSpec((FD, H0), lambda i: (0, 0)),
        pl.BlockSpec((C, H0), lambda i: (0, 0)),
        pl.BlockSpec((1, H0), lambda i: (0, 0)),
        pl.BlockSpec((H0, H1), lambda i: (0, 0)),
        pl.BlockSpec((1, H1), lambda i: (0, 0)),
        pl.BlockSpec((H1, H2), lambda i: (0, 0)),
        pl.BlockSpec((1, H2), lambda i: (0, 0)),
    ],
    out_specs=pl.BlockSpec((BM, H2), lambda i: (i, 0)),
    out_shape=jax.ShapeDtypeStruct((B, H2), jnp.float32),
)


def kernel(continuous, categorical_indices, tables, W0, b0, W1, b1, W2, b2):
    offsets = (jnp.arange(F, dtype=jnp.int32) * V)[None, :]
    flat = categorical_indices + offsets
    t3 = flat.reshape(NW, BPW, F).transpose(0, 2, 1)
    quad = (t3 >> 2).reshape(-1)
    qb = ((t3 & 3) << 5).reshape(-1)
    tab128 = _k1(tables)
    emb = _k2(tab128, quad, qb)
    return _mlp(emb, continuous.astype(jnp.float32),
                W0[:FD], W0[FD:],
                b0.reshape(1, H0), W1, b1.reshape(1, H1),
                W2, b2.reshape(1, H2))


# speedup vs baseline: 1.3370x; 1.3370x over previous
import functools

import jax
import jax.numpy as jnp
from jax import lax
from jax.experimental import pallas as pl
from jax.experimental.pallas import tpu as pltpu
from jax.experimental.pallas import tpu_sc as plsc

B = 16384
F = 26
V = 100000
D = 32
C = 16
H0, H1, H2 = 512, 256, 128

NC, NS = 2, 16
NW = NC * NS
TOTAL = B * F
CHUNK = 128
N_CHUNKS = TOTAL // CHUNK
CPW = N_CHUNKS // NW
RING = 8
PF = 4

_mesh = plsc.VectorSubcoreMesh(core_axis_name="c", subcore_axis_name="s")


@functools.partial(
    pl.kernel,
    out_type=jax.ShapeDtypeStruct((TOTAL, D), jnp.float32),
    mesh=_mesh,
    scratch_types=(
        [pltpu.VMEM((CPW, CHUNK), jnp.int32)]
        + [pltpu.VMEM((CHUNK, D), jnp.float32)] * RING
        + [pltpu.SemaphoreType.DMA] * (2 * RING)
    ),
    compiler_params=pltpu.CompilerParams(use_tc_tiling_on_sc=False),
)
def _sc_gather(idx_hbm, tables_hbm, out_hbm, *s):
    idx_v = s[0]
    bufs = s[1:1 + RING]
    srs = s[1 + RING:1 + 2 * RING]
    sws = s[1 + 2 * RING:1 + 3 * RING]
    wid = lax.axis_index("s") * NC + lax.axis_index("c")
    c0 = wid * CPW
    pltpu.sync_copy(idx_hbm.at[pl.ds(c0, CPW), :], idx_v)

    def gissue(c, slot):
        pltpu.async_copy(tables_hbm.at[idx_v.at[c]], bufs[slot], srs[slot])

    for st in range(PF):
        gissue(st, st)

    def blk_body(blk, carry):
        for st in range(RING):
            c = blk * RING + st
            buf, sr, sw = bufs[st], srs[st], sws[st]
            pltpu.make_async_copy(
                tables_hbm.at[idx_v.at[c]], buf, sr).wait()
            pltpu.async_copy(
                buf, out_hbm.at[pl.ds((c0 + c) * CHUNK, CHUNK), :], sw)
            nslot = (st + PF) % RING

            @pl.when(c + PF >= RING)
            def _():
                pltpu.make_async_copy(
                    bufs[nslot], out_hbm.at[pl.ds(0, CHUNK), :],
                    sws[nslot]).wait()

            @pl.when(c + PF < CPW)
            def _():
                gissue(c + PF, nslot)
        return carry

    lax.fori_loop(0, CPW // RING, blk_body, 0)
    for st in range(PF):
        slot = (CPW - PF + st) % RING
        pltpu.make_async_copy(
            bufs[slot], out_hbm.at[pl.ds(0, CHUNK), :], sws[slot]).wait()


BM = 512


def _mlp_body(emb_ref, cont_ref, w0a_ref, w0b_ref, b0_ref, w1_ref, b1_ref,
              w2_ref, b2_ref, out_ref):
    h0 = jnp.dot(emb_ref[...], w0a_ref[...], preferred_element_type=jnp.float32)
    h0 += jnp.dot(cont_ref[...], w0b_ref[...], preferred_element_type=jnp.float32)
    h0 = jnp.maximum(h0 + b0_ref[...], 0.0)
    h1 = jnp.maximum(
        jnp.dot(h0, w1_ref[...], preferred_element_type=jnp.float32) + b1_ref[...], 0.0)
    out_ref[...] = jnp.maximum(
        jnp.dot(h1, w2_ref[...], preferred_element_type=jnp.float32) + b2_ref[...], 0.0)


_mlp = pl.pallas_call(
    _mlp_body,
    grid=(B // BM,),
    in_specs=[
        pl.BlockSpec((BM, F * D), lambda i: (i, 0)),
        pl.BlockSpec((BM, C), lambda i: (i, 0)),
        pl.BlockSpec((F * D, H0), lambda i: (0, 0)),
        pl.BlockSpec((C, H0), lambda i: (0, 0)),
        pl.BlockSpec((1, H0), lambda i: (0, 0)),
        pl.BlockSpec((H0, H1), lambda i: (0, 0)),
        pl.BlockSpec((1, H1), lambda i: (0, 0)),
        pl.BlockSpec((H1, H2), lambda i: (0, 0)),
        pl.BlockSpec((1, H2), lambda i: (0, 0)),
    ],
    out_specs=pl.BlockSpec((BM, H2), lambda i: (i, 0)),
    out_shape=jax.ShapeDtypeStruct((B, H2), jnp.float32),
)


def kernel(continuous, categorical_indices, tables, W0, b0, W1, b1, W2, b2):
    offsets = (jnp.arange(F, dtype=jnp.int32) * V)[None, :]
    flat_idx = (categorical_indices + offsets).reshape(N_CHUNKS, CHUNK)
    emb_flat = _sc_gather(flat_idx, tables)
    emb = emb_flat.reshape(B, F * D)
    return _mlp(emb, continuous.astype(jnp.float32),
                W0[:F * D], W0[F * D:],
                b0.reshape(1, H0), W1, b1.reshape(1, H1),
                W2, b2.reshape(1, H2))
